# baseline (speedup 1.0000x reference)
import functools

import jax
import jax.numpy as jnp
from jax import lax
from jax.experimental import pallas as pl
from jax.experimental.pallas import tpu as pltpu
from jax.experimental.pallas import tpu_sc as plsc

_N_GRAPHS = 512
_ROW_BLOCK = 16
_SC_ROWS = 256
_N_WORKERS = 32
_W = 6272
_W_SHORT = 6144
_SC_COLS = 6 * _W + 2 * _W_SHORT


def _tc_block(neuron_ref, concept_ref, coefs_ref, out_ref):
    x = neuron_ref[...]
    cm = concept_ref[...]
    c = coefs_ref[...]
    c0 = c[:, 0:1]
    c1 = c[:, 1:2]
    c2 = c[:, 2:3]
    c3 = c[:, 3:4]
    t = c0 + x * (c1 + x * (c2 + x * c3))
    d = t - cm
    sq = d * d
    out_ref[...] = jnp.sum(sq, axis=1, keepdims=True) * (1.0 / _N_GRAPHS)


def _sc_worker(neuron_hbm, concept_hbm, coefs_hbm, out_hbm,
               xbuf, cmbuf, coefs_v, res_v):
    row0 = neuron_hbm.shape[0] - _SC_ROWS
    wid = lax.axis_index("s") * 2 + lax.axis_index("c")
    g = wid
    row8 = pl.multiple_of(row0 + g * 8, 8)
    rows = pl.ds(row8, 8)
    pltpu.sync_copy(coefs_hbm.at[rows], coefs_v)

    accs = [jnp.zeros((16,), jnp.float32) for _ in range(8)]
    widths = [_W] * 6 + [_W_SHORT] * 2
    starts = [sum(widths[:c]) for c in range(8)]
    for chunk in range(8):
        w = widths[chunk]
        start = starts[chunk]
        pltpu.sync_copy(neuron_hbm.at[rows, pl.ds(start, w)],
                        xbuf.at[:, pl.ds(0, w)])
        pltpu.sync_copy(concept_hbm.at[rows, pl.ds(start, w)],
                        cmbuf.at[:, pl.ds(0, w)])
        for r in range(8):
            cv = coefs_v[r, :]
            c0 = cv[0]
            c1 = cv[1]
            c2 = cv[2]
            c3 = cv[3]

            def body(j, a, r=r, c0=c0, c1=c1, c2=c2, c3=c3):
                x = xbuf[r, pl.ds(j * 16, 16)]
                cm = cmbuf[r, pl.ds(j * 16, 16)]
                t = c0 + x * (c1 + x * (c2 + x * c3))
                d = t - cm
                return a + d * d

            accs[r] = lax.fori_loop(0, w // 16, body, accs[r], unroll=8)

    for r in range(8):
        res_v[r, :] = accs[r]
    pltpu.sync_copy(res_v, out_hbm.at[wid])


def kernel(neuron_mat, concept_mat, coefs, graph_idxs):
    del graph_idxs
    n_concepts, n_nodes = neuron_mat.shape
    n_tc_rows = n_concepts - _SC_ROWS

    sc_call = functools.partial(
        pl.kernel,
        mesh=plsc.VectorSubcoreMesh(core_axis_name="c", subcore_axis_name="s"),
        out_type=jax.ShapeDtypeStruct((_N_WORKERS, 8, 16), jnp.float32),
        scratch_types=[
            pltpu.VMEM((8, _W), jnp.float32),
            pltpu.VMEM((8, _W), jnp.float32),
            pltpu.VMEM((8, 16), jnp.float32),
            pltpu.VMEM((8, 16), jnp.float32),
        ],
    )(_sc_worker)
    coefs_pad = jnp.pad(coefs, ((0, 0), (0, 16 - coefs.shape[1])))
    sc_parts = sc_call(neuron_mat, concept_mat, coefs_pad)

    sc_out = sc_parts.sum(axis=-1).reshape(_SC_ROWS)

    sl_x = lax.slice(neuron_mat, (n_tc_rows, _SC_COLS), (n_concepts, n_nodes))
    sl_cm = lax.slice(concept_mat, (n_tc_rows, _SC_COLS),
                      (n_concepts, n_nodes))
    sl_c = lax.slice(coefs, (n_tc_rows, 0), (n_concepts, coefs.shape[1]))
    c0, c1, c2, c3 = (sl_c[:, k:k + 1] for k in range(4))
    sl_t = c0 + sl_x * (c1 + sl_x * (c2 + sl_x * c3)) - sl_cm
    sc_out = (sc_out + jnp.sum(sl_t * sl_t, axis=1)) * (1.0 / _N_GRAPHS)

    return sc_out


# speedup vs baseline: 1.3829x; 1.3829x over previous
import functools

import jax
import jax.numpy as jnp
from jax import lax
from jax.experimental import pallas as pl
from jax.experimental.pallas import tpu as pltpu
from jax.experimental.pallas import tpu_sc as plsc

_N_GRAPHS = 512
_ROW_BLOCK = 16
_SC_ROWS = 64
_N_WORKERS = 32
_W = 6272
_W_SHORT = 6144
_SC_COLS = 6 * _W + 2 * _W_SHORT


def _tc_block(neuron_ref, concept_ref, coefs_ref, out_ref):
    x = neuron_ref[...]
    cm = concept_ref[...]
    c = coefs_ref[...]
    c0 = c[:, 0:1]
    c1 = c[:, 1:2]
    c2 = c[:, 2:3]
    c3 = c[:, 3:4]
    t = c0 + x * (c1 + x * (c2 + x * c3))
    d = t - cm
    sq = d * d
    out_ref[...] = jnp.sum(sq, axis=1, keepdims=True) * (1.0 / _N_GRAPHS)


def _sc_worker(neuron_hbm, concept_hbm, coefs_hbm, out_hbm,
               xbuf, cmbuf, coefs_v, res_v):
    row0 = neuron_hbm.shape[0] - _SC_ROWS
    wid = lax.axis_index("s") * 2 + lax.axis_index("c")
    g = wid // 4
    q = wid % 4
    row8 = pl.multiple_of(row0 + g * 8, 8)
    rows = pl.ds(row8, 8)
    pltpu.sync_copy(coefs_hbm.at[rows], coefs_v)

    accs = [jnp.zeros((16,), jnp.float32) for _ in range(8)]
    for slot in range(2):
        chunk = q + 4 * slot
        is_short = chunk >= 6
        start = pl.multiple_of(
            jnp.minimum(chunk, 6) * _W + jnp.maximum(chunk - 6, 0) * _W_SHORT,
            128)

        @pl.when(jnp.logical_not(is_short))
        def _():
            pltpu.sync_copy(neuron_hbm.at[rows, pl.ds(start, _W)], xbuf)
            pltpu.sync_copy(concept_hbm.at[rows, pl.ds(start, _W)], cmbuf)

        @pl.when(is_short)
        def _():
            pltpu.sync_copy(neuron_hbm.at[rows, pl.ds(start, _W_SHORT)],
                            xbuf.at[:, pl.ds(0, _W_SHORT)])
            pltpu.sync_copy(concept_hbm.at[rows, pl.ds(start, _W_SHORT)],
                            cmbuf.at[:, pl.ds(0, _W_SHORT)])

        n_it = jnp.where(is_short, _W_SHORT // 16, _W // 16)
        for r in range(8):
            cv = coefs_v[r, :]
            c0 = cv[0]
            c1 = cv[1]
            c2 = cv[2]
            c3 = cv[3]

            def body(j, a, r=r, c0=c0, c1=c1, c2=c2, c3=c3):
                x = xbuf[r, pl.ds(j * 16, 16)]
                cm = cmbuf[r, pl.ds(j * 16, 16)]
                t = c0 + x * (c1 + x * (c2 + x * c3))
                d = t - cm
                return a + jnp.where(j < n_it, d * d, 0.0)

            accs[r] = lax.fori_loop(0, _W // 16, body, accs[r], unroll=8)

    for r in range(8):
        res_v[r, :] = accs[r]
    pltpu.sync_copy(res_v, out_hbm.at[wid])


def kernel(neuron_mat, concept_mat, coefs, graph_idxs):
    del graph_idxs
    n_concepts, n_nodes = neuron_mat.shape
    n_tc_rows = n_concepts - _SC_ROWS
    nr = n_tc_rows // _ROW_BLOCK
    assert nr * _ROW_BLOCK == n_tc_rows

    sc_call = functools.partial(
        pl.kernel,
        mesh=plsc.VectorSubcoreMesh(core_axis_name="c", subcore_axis_name="s"),
        out_type=jax.ShapeDtypeStruct((_N_WORKERS, 8, 16), jnp.float32),
        scratch_types=[
            pltpu.VMEM((8, _W), jnp.float32),
            pltpu.VMEM((8, _W), jnp.float32),
            pltpu.VMEM((8, 16), jnp.float32),
            pltpu.VMEM((8, 16), jnp.float32),
        ],
    )(_sc_worker)
    coefs_pad = jnp.pad(coefs, ((0, 0), (0, 16 - coefs.shape[1])))
    sc_parts = sc_call(neuron_mat, concept_mat, coefs_pad)

    tc_out = pl.pallas_call(
        _tc_block,
        grid=(nr,),
        in_specs=[
            pl.BlockSpec((_ROW_BLOCK, n_nodes), lambda i: (i, 0)),
            pl.BlockSpec((_ROW_BLOCK, n_nodes), lambda i: (i, 0)),
            pl.BlockSpec((_ROW_BLOCK, coefs.shape[1]), lambda i: (i, 0)),
        ],
        out_specs=pl.BlockSpec((_ROW_BLOCK, 1), lambda i: (i, 0)),
        out_shape=jax.ShapeDtypeStruct((n_tc_rows, 1), jnp.float32),
        compiler_params=pltpu.CompilerParams(
            dimension_semantics=("parallel",)),
    )(neuron_mat, concept_mat, coefs)

    sc_out = (sc_parts.sum(axis=-1).reshape(8, 4, 8).sum(axis=1)
              .reshape(_SC_ROWS))

    sl_x = lax.slice(neuron_mat, (n_tc_rows, _SC_COLS), (n_concepts, n_nodes))
    sl_cm = lax.slice(concept_mat, (n_tc_rows, _SC_COLS),
                      (n_concepts, n_nodes))
    sl_c = lax.slice(coefs, (n_tc_rows, 0), (n_concepts, coefs.shape[1]))
    c0, c1, c2, c3 = (sl_c[:, k:k + 1] for k in range(4))
    sl_t = c0 + sl_x * (c1 + sl_x * (c2 + sl_x * c3)) - sl_cm
    sc_out = (sc_out + jnp.sum(sl_t * sl_t, axis=1)) * (1.0 / _N_GRAPHS)

    return jnp.concatenate([tc_out[:, 0], sc_out])


# speedup vs baseline: 1.5764x; 1.1399x over previous
import jax
import jax.numpy as jnp
from jax.experimental import pallas as pl
from jax.experimental.pallas import tpu as pltpu

_N_GRAPHS = 512
_K = 4
_SUB = 8
_STEP = _K * _SUB


def _shaper_block(*refs):
    (n0, n1, n2, n3, m0, m1, m2, m3, coefs_ref, out_ref) = refs
    for k, (nk, mk) in enumerate(((n0, m0), (n1, m1), (n2, m2), (n3, m3))):
        x = nk[...]
        cm = mk[...]
        c = coefs_ref[pl.ds(k * _SUB, _SUB), :]
        c0 = c[:, 0:1]
        c1 = c[:, 1:2]
        c2 = c[:, 2:3]
        c3 = c[:, 3:4]
        t = c0 + x * (c1 + x * (c2 + x * c3))
        d = t - cm
        sq = d * d
        out_ref[pl.ds(k * _SUB, _SUB), :] = (
            jnp.sum(sq, axis=1, keepdims=True) * (1.0 / _N_GRAPHS))


def kernel(neuron_mat, concept_mat, coefs, graph_idxs):
    del graph_idxs
    n_concepts, n_nodes = neuron_mat.shape
    nr = n_concepts // _STEP
    assert nr * _STEP == n_concepts

    def sub_spec(k):
        return pl.BlockSpec((_SUB, n_nodes), lambda i, k=k: (_K * i + k, 0))

    out = pl.pallas_call(
        _shaper_block,
        grid=(nr,),
        in_specs=(
            [sub_spec(k) for k in range(_K)]
            + [sub_spec(k) for k in range(_K)]
            + [pl.BlockSpec((_STEP, coefs.shape[1]), lambda i: (i, 0))]
        ),
        out_specs=pl.BlockSpec((_STEP, 1), lambda i: (i, 0)),
        out_shape=jax.ShapeDtypeStruct((n_concepts, 1), jnp.float32),
        compiler_params=pltpu.CompilerParams(
            dimension_semantics=("parallel",)),
    )(*([neuron_mat] * _K + [concept_mat] * _K + [coefs]))
    return out[:, 0]
